# Initial kernel scaffold; baseline (speedup 1.0000x reference)
import jax
import jax.numpy as jnp
from jax.experimental import pallas as pl


def kernel(h_t, W_mix, W_gate, W1, b1, W2, b2):
    raise NotImplementedError("write your pallas kernel here")


# speedup vs baseline: 1.5033x; 1.5033x over previous
import functools

import jax
import jax.numpy as jnp
from jax.experimental import pallas as pl
from jax.experimental.pallas import tpu as pltpu

NUM_EXPERTS = 8
TOP_K = 2
T = 2048
D = 768
FF = 2048

MG_TILE = 256
RT_TILE = 256
EX_TILE = 512


def _mixgate_body(x_ref, wmix_ref, wgate_ref, h_ref, logits_ref):
    x = x_ref[...]
    xm = jnp.dot(x, wmix_ref[...], preferred_element_type=jnp.float32)
    h = x + jnp.tanh(xm)
    h_ref[...] = h
    logits_ref[...] = jnp.dot(h, wgate_ref[...], preferred_element_type=jnp.float32)


def _route_body(logits_ref, probs_ref, m_ref, zsum_ref, lbl_ref, load_ref, cnt_ref):
    i = pl.program_id(0)
    n = pl.num_programs(0)
    logits = logits_ref[...]
    mx = jnp.max(logits, axis=1, keepdims=True)
    ex = jnp.exp(logits - mx)
    den = jnp.sum(ex, axis=1, keepdims=True)
    probs = ex / den
    probs_ref[...] = probs

    e_iota = jax.lax.broadcasted_iota(jnp.int32, probs.shape, 1)
    p1 = jnp.max(probs, axis=1, keepdims=True)
    is1 = probs == p1
    i1 = jnp.min(jnp.where(is1, e_iota, NUM_EXPERTS), axis=1, keepdims=True)
    sel1 = e_iota == i1
    pm = jnp.where(sel1, -jnp.inf, probs)
    p2 = jnp.max(pm, axis=1, keepdims=True)
    is2 = pm == p2
    i2 = jnp.min(jnp.where(is2, e_iota, NUM_EXPERTS), axis=1, keepdims=True)
    sel2 = e_iota == i2
    wsum = p1 + p2
    m = jnp.where(sel1, p1 / wsum, jnp.where(sel2, p2 / wsum, 0.0))
    m_ref[...] = m

    z = jnp.log(den[:, 0]) + mx[:, 0]
    z2 = jnp.sum(z * z).reshape(1, 1)
    psum = jnp.sum(probs, axis=0, keepdims=True)
    csum = jnp.sum((sel1 | sel2).astype(jnp.float32), axis=0, keepdims=True)

    @pl.when(i == 0)
    def _init():
        zsum_ref[...] = z2
        load_ref[...] = psum
        cnt_ref[...] = csum

    @pl.when(i > 0)
    def _acc():
        zsum_ref[...] += z2
        load_ref[...] += psum
        cnt_ref[...] += csum

    @pl.when(i == n - 1)
    def _fin():
        zsum_ref[...] = zsum_ref[...] / T
        load = load_ref[...] / T
        load_ref[...] = load
        frac = cnt_ref[...] / (T * TOP_K)
        lbl_ref[...] = (NUM_EXPERTS * jnp.sum(frac * load)).reshape(1, 1)

    @pl.when(i < n - 1)
    def _nofin():
        lbl_ref[...] = jnp.zeros((1, 1), jnp.float32)


def _expert_body(h_ref, w1_ref, b1_ref, w2_ref, b2_ref, mt_ref, out_ref):
    e = pl.program_id(1)
    h = h_ref[...]
    hid = jnp.dot(h, w1_ref[0], preferred_element_type=jnp.float32) + b1_ref[0]
    hid = jax.nn.gelu(hid)
    o = jnp.dot(hid, w2_ref[0], preferred_element_type=jnp.float32) + b2_ref[0]
    w = mt_ref[0, 0, :].reshape(-1, 1)
    contrib = w * o

    @pl.when(e == 0)
    def _init():
        out_ref[...] = contrib

    @pl.when(e > 0)
    def _acc():
        out_ref[...] += contrib


def kernel(h_t, W_mix, W_gate, W1, b1, W2, b2):
    x = h_t.reshape(T, D)

    h, gate_logits = pl.pallas_call(
        _mixgate_body,
        grid=(T // MG_TILE,),
        in_specs=[
            pl.BlockSpec((MG_TILE, D), lambda i: (i, 0)),
            pl.BlockSpec((D, D), lambda i: (0, 0)),
            pl.BlockSpec((D, NUM_EXPERTS), lambda i: (0, 0)),
        ],
        out_specs=[
            pl.BlockSpec((MG_TILE, D), lambda i: (i, 0)),
            pl.BlockSpec((MG_TILE, NUM_EXPERTS), lambda i: (i, 0)),
        ],
        out_shape=[
            jax.ShapeDtypeStruct((T, D), jnp.float32),
            jax.ShapeDtypeStruct((T, NUM_EXPERTS), jnp.float32),
        ],
    )(x, W_mix, W_gate)

    probs, m, zloss, lbl, load, cnt = pl.pallas_call(
        _route_body,
        grid=(T // RT_TILE,),
        in_specs=[pl.BlockSpec((RT_TILE, NUM_EXPERTS), lambda i: (i, 0))],
        out_specs=[
            pl.BlockSpec((RT_TILE, NUM_EXPERTS), lambda i: (i, 0)),
            pl.BlockSpec((RT_TILE, NUM_EXPERTS), lambda i: (i, 0)),
            pl.BlockSpec((1, 1), lambda i: (0, 0)),
            pl.BlockSpec((1, 1), lambda i: (0, 0)),
            pl.BlockSpec((1, NUM_EXPERTS), lambda i: (0, 0)),
            pl.BlockSpec((1, NUM_EXPERTS), lambda i: (0, 0)),
        ],
        out_shape=[
            jax.ShapeDtypeStruct((T, NUM_EXPERTS), jnp.float32),
            jax.ShapeDtypeStruct((T, NUM_EXPERTS), jnp.float32),
            jax.ShapeDtypeStruct((1, 1), jnp.float32),
            jax.ShapeDtypeStruct((1, 1), jnp.float32),
            jax.ShapeDtypeStruct((1, NUM_EXPERTS), jnp.float32),
            jax.ShapeDtypeStruct((1, NUM_EXPERTS), jnp.float32),
        ],
    )(gate_logits)

    mt = m.T.reshape(NUM_EXPERTS, 1, T)
    b1r = b1.reshape(NUM_EXPERTS, 1, FF)
    b2r = b2.reshape(NUM_EXPERTS, 1, D)

    final = pl.pallas_call(
        _expert_body,
        grid=(T // EX_TILE, NUM_EXPERTS),
        in_specs=[
            pl.BlockSpec((EX_TILE, D), lambda t, e: (t, 0)),
            pl.BlockSpec((1, D, FF), lambda t, e: (e, 0, 0)),
            pl.BlockSpec((1, 1, FF), lambda t, e: (e, 0, 0)),
            pl.BlockSpec((1, FF, D), lambda t, e: (e, 0, 0)),
            pl.BlockSpec((1, 1, D), lambda t, e: (e, 0, 0)),
            pl.BlockSpec((1, 1, EX_TILE), lambda t, e: (e, 0, t)),
        ],
        out_specs=pl.BlockSpec((EX_TILE, D), lambda t, e: (t, 0)),
        out_shape=jax.ShapeDtypeStruct((T, D), jnp.float32),
    )(h, W1, b1r, W2, b2r, mt)

    final_hidden_states = final.reshape(1, T, D)
    z_loss = zloss.reshape(())
    load_balancing_loss = lbl.reshape(())
    expert_load = load.reshape(NUM_EXPERTS)
    expert_token_counts = cnt.reshape(NUM_EXPERTS).astype(jnp.int32)
    return (gate_logits, probs, final_hidden_states, z_loss,
            load_balancing_loss, expert_load, expert_token_counts)


# speedup vs baseline: 1.6856x; 1.1212x over previous
import functools

import jax
import jax.numpy as jnp
from jax.experimental import pallas as pl
from jax.experimental.pallas import tpu as pltpu

NUM_EXPERTS = 8
TOP_K = 2
T = 2048
D = 768
FF = 2048

MG_TILE = 256
RT_TILE = 256
EX_TILE = 1024


def _mixgate_body(x_ref, wmix_ref, wgate_ref, h_ref, logits_ref):
    x = x_ref[...]
    xm = jnp.dot(x, wmix_ref[...], preferred_element_type=jnp.float32)
    h = x + jnp.tanh(xm)
    h_ref[...] = h
    logits_ref[...] = jnp.dot(h, wgate_ref[...], preferred_element_type=jnp.float32)


def _route_body(logits_ref, probs_ref, m_ref, zsum_ref, lbl_ref, load_ref, cnt_ref):
    i = pl.program_id(0)
    n = pl.num_programs(0)
    logits = logits_ref[...]
    mx = jnp.max(logits, axis=1, keepdims=True)
    ex = jnp.exp(logits - mx)
    den = jnp.sum(ex, axis=1, keepdims=True)
    probs = ex / den
    probs_ref[...] = probs

    e_iota = jax.lax.broadcasted_iota(jnp.int32, probs.shape, 1)
    p1 = jnp.max(probs, axis=1, keepdims=True)
    is1 = probs == p1
    i1 = jnp.min(jnp.where(is1, e_iota, NUM_EXPERTS), axis=1, keepdims=True)
    sel1 = e_iota == i1
    pm = jnp.where(sel1, -jnp.inf, probs)
    p2 = jnp.max(pm, axis=1, keepdims=True)
    is2 = pm == p2
    i2 = jnp.min(jnp.where(is2, e_iota, NUM_EXPERTS), axis=1, keepdims=True)
    sel2 = e_iota == i2
    wsum = p1 + p2
    m = jnp.where(sel1, p1 / wsum, jnp.where(sel2, p2 / wsum, 0.0))
    m_ref[...] = m

    z = jnp.log(den[:, 0]) + mx[:, 0]
    z2 = jnp.sum(z * z).reshape(1, 1)
    psum = jnp.sum(probs, axis=0, keepdims=True)
    csum = jnp.sum((sel1 | sel2).astype(jnp.float32), axis=0, keepdims=True)

    @pl.when(i == 0)
    def _init():
        zsum_ref[...] = z2
        load_ref[...] = psum
        cnt_ref[...] = csum

    @pl.when(i > 0)
    def _acc():
        zsum_ref[...] += z2
        load_ref[...] += psum
        cnt_ref[...] += csum

    @pl.when(i == n - 1)
    def _fin():
        zsum_ref[...] = zsum_ref[...] / T
        load = load_ref[...] / T
        load_ref[...] = load
        frac = cnt_ref[...] / (T * TOP_K)
        lbl_ref[...] = (NUM_EXPERTS * jnp.sum(frac * load)).reshape(1, 1)

    @pl.when(i < n - 1)
    def _nofin():
        lbl_ref[...] = jnp.zeros((1, 1), jnp.float32)


def _expert_body(h_ref, w1_ref, b1_ref, w2_ref, b2_ref, mt_ref, out_ref):
    e = pl.program_id(1)
    h = h_ref[...]
    hid = jnp.dot(h, w1_ref[0], preferred_element_type=jnp.float32) + b1_ref[0]
    hid = jax.nn.gelu(hid)
    o = jnp.dot(hid, w2_ref[0], preferred_element_type=jnp.float32) + b2_ref[0]
    w = mt_ref[0, 0, :].reshape(-1, 1)
    contrib = w * o

    @pl.when(e == 0)
    def _init():
        out_ref[...] = contrib

    @pl.when(e > 0)
    def _acc():
        out_ref[...] += contrib


def kernel(h_t, W_mix, W_gate, W1, b1, W2, b2):
    x = h_t.reshape(T, D)

    h, gate_logits = pl.pallas_call(
        _mixgate_body,
        grid=(T // MG_TILE,),
        in_specs=[
            pl.BlockSpec((MG_TILE, D), lambda i: (i, 0)),
            pl.BlockSpec((D, D), lambda i: (0, 0)),
            pl.BlockSpec((D, NUM_EXPERTS), lambda i: (0, 0)),
        ],
        out_specs=[
            pl.BlockSpec((MG_TILE, D), lambda i: (i, 0)),
            pl.BlockSpec((MG_TILE, NUM_EXPERTS), lambda i: (i, 0)),
        ],
        out_shape=[
            jax.ShapeDtypeStruct((T, D), jnp.float32),
            jax.ShapeDtypeStruct((T, NUM_EXPERTS), jnp.float32),
        ],
    )(x, W_mix, W_gate)

    probs, m, zloss, lbl, load, cnt = pl.pallas_call(
        _route_body,
        grid=(T // RT_TILE,),
        in_specs=[pl.BlockSpec((RT_TILE, NUM_EXPERTS), lambda i: (i, 0))],
        out_specs=[
            pl.BlockSpec((RT_TILE, NUM_EXPERTS), lambda i: (i, 0)),
            pl.BlockSpec((RT_TILE, NUM_EXPERTS), lambda i: (i, 0)),
            pl.BlockSpec((1, 1), lambda i: (0, 0)),
            pl.BlockSpec((1, 1), lambda i: (0, 0)),
            pl.BlockSpec((1, NUM_EXPERTS), lambda i: (0, 0)),
            pl.BlockSpec((1, NUM_EXPERTS), lambda i: (0, 0)),
        ],
        out_shape=[
            jax.ShapeDtypeStruct((T, NUM_EXPERTS), jnp.float32),
            jax.ShapeDtypeStruct((T, NUM_EXPERTS), jnp.float32),
            jax.ShapeDtypeStruct((1, 1), jnp.float32),
            jax.ShapeDtypeStruct((1, 1), jnp.float32),
            jax.ShapeDtypeStruct((1, NUM_EXPERTS), jnp.float32),
            jax.ShapeDtypeStruct((1, NUM_EXPERTS), jnp.float32),
        ],
    )(gate_logits)

    mt = m.T.reshape(NUM_EXPERTS, 1, T)
    b1r = b1.reshape(NUM_EXPERTS, 1, FF)
    b2r = b2.reshape(NUM_EXPERTS, 1, D)

    final = pl.pallas_call(
        _expert_body,
        grid=(T // EX_TILE, NUM_EXPERTS),
        in_specs=[
            pl.BlockSpec((EX_TILE, D), lambda t, e: (t, 0)),
            pl.BlockSpec((1, D, FF), lambda t, e: (e, 0, 0)),
            pl.BlockSpec((1, 1, FF), lambda t, e: (e, 0, 0)),
            pl.BlockSpec((1, FF, D), lambda t, e: (e, 0, 0)),
            pl.BlockSpec((1, 1, D), lambda t, e: (e, 0, 0)),
            pl.BlockSpec((1, 1, EX_TILE), lambda t, e: (e, 0, t)),
        ],
        out_specs=pl.BlockSpec((EX_TILE, D), lambda t, e: (t, 0)),
        out_shape=jax.ShapeDtypeStruct((T, D), jnp.float32),
    )(h, W1, b1r, W2, b2r, mt)

    final_hidden_states = final.reshape(1, T, D)
    z_loss = zloss.reshape(())
    load_balancing_loss = lbl.reshape(())
    expert_load = load.reshape(NUM_EXPERTS)
    expert_token_counts = cnt.reshape(NUM_EXPERTS).astype(jnp.int32)
    return (gate_logits, probs, final_hidden_states, z_loss,
            load_balancing_loss, expert_load, expert_token_counts)
